# baseline (speedup 1.0000x reference)
import functools

import jax
import jax.numpy as jnp
from jax import lax
from jax.experimental import pallas as pl
from jax.experimental.pallas import tpu as pltpu
from jax.experimental.pallas import tpu_sc as plsc

MAX_CTX = 32768
DIM = 2048

_HALF_ROWS = MAX_CTX // 2
_WORKERS = 32
_PER_W = MAX_CTX // _WORKERS

_T_CH = 16
_T_NCH = _PER_W // _T_CH
_S_CH = 32
_S_NCH = _PER_W // _S_CH

_mesh = plsc.VectorSubcoreMesh(core_axis_name="c", subcore_axis_name="s")


@functools.partial(
    pl.kernel,
    out_type=jax.ShapeDtypeStruct((MAX_CTX, DIM), jnp.float32),
    mesh=_mesh,
    scratch_types=[
        pltpu.VMEM((2, _T_CH, DIM), jnp.float32),
        pltpu.VMEM_SHARED((8, 2, _S_CH, DIM), jnp.float32),
        pltpu.SemaphoreType.DMA((2,)),
        pltpu.SemaphoreType.DMA((2,)),
    ],
)
def _sc_copy(mem_hbm, h_hbm, out_hbm, tbuf, sbuf, rsem, wsem):
    cid = lax.axis_index("c")
    sid = lax.axis_index("s")
    wid = cid * 16 + sid
    base = wid * _PER_W

    def stripe(src_ref, src_base, bufsl, ch, nch):
        def read(c, slot):
            return pltpu.make_async_copy(
                src_ref.at[pl.ds(src_base + c * ch, ch), :],
                bufsl.at[slot], rsem.at[slot])

        def write(c, slot):
            return pltpu.make_async_copy(
                bufsl.at[slot],
                out_hbm.at[pl.ds(base + c * ch, ch), :], wsem.at[slot])

        read(0, 0).start()

        def step(c, _):
            slot = lax.rem(c, 2)
            nslot = lax.rem(c + 1, 2)
            read(c, slot).wait()

            @pl.when(c >= 1)
            def _():
                write(c - 1, nslot).wait()

            @pl.when(c + 1 < nch)
            def _():
                read(c + 1, nslot).start()

            write(c, slot).start()
            return 0

        lax.fori_loop(0, nch, step, 0)
        write(nch - 1, (nch - 1) % 2).wait()

    def both_paths(src_ref, src_base):
        @pl.when(sid < 8)
        def _():
            stripe(src_ref, src_base, tbuf, _T_CH, _T_NCH)

        @pl.when(sid >= 8)
        def _():
            stripe(src_ref, src_base, sbuf.at[sid - 8], _S_CH, _S_NCH)

    @pl.when(wid < _WORKERS // 2)
    def _():
        both_paths(mem_hbm, base + _HALF_ROWS)

    @pl.when(wid >= _WORKERS // 2)
    def _():
        both_paths(h_hbm, base - _HALF_ROWS)


def kernel(h, mem):
    B, L, D = h.shape
    flat = h.reshape(B * L, D)
    new_mem = _sc_copy(mem, flat)
    return h, new_mem
